# baseline (speedup 1.0000x reference)
import jax
import jax.numpy as jnp
from jax.experimental import pallas as pl
from jax.experimental.pallas import tpu as pltpu

_PREC = jax.lax.Precision.HIGHEST


def _prep_kernel(adj_ref, ar_ref, dv_ref):
    a = adj_ref[...]
    n = a.shape[0]
    row = jax.lax.broadcasted_iota(jnp.int32, (n, n), 0)
    col = jax.lax.broadcasted_iota(jnp.int32, (n, n), 1)
    a = a + jnp.where(row == col, 1.0, 0.0).astype(a.dtype)
    d = jnp.sum(a, axis=1, keepdims=True)
    dinv = jax.lax.rsqrt(d)
    ar_ref[...] = a * dinv
    dv_ref[...] = jnp.broadcast_to(dinv, dv_ref.shape)


def _layernorm(h, g, be, eps=1e-5):
    mu = jnp.mean(h, axis=-1, keepdims=True)
    c = h - mu
    var = jnp.mean(c * c, axis=-1, keepdims=True)
    return c * jax.lax.rsqrt(var + eps) * g + be


def _gcn_kernel(ar_ref, dv_ref, x_ref,
                w1_ref, b1_ref, g1_ref, be1_ref,
                w2_ref, b2_ref, g2_ref, be2_ref,
                wo_ref, bo_ref, go_ref, beo_ref, o_ref):
    ar = ar_ref[...]
    dv = dv_ref[...][:, :x_ref.shape[-1]]
    xx = x_ref[0]
    h = xx
    for (w_r, b_r, g_r, be_r) in ((w1_ref, b1_ref, g1_ref, be1_ref),
                                  (w2_ref, b2_ref, g2_ref, be2_ref)):
        hn = _layernorm(h, g_r[...], be_r[...])
        m = jnp.dot(ar, dv * hn, precision=_PREC,
                    preferred_element_type=jnp.float32)
        h = h + jax.nn.relu(jnp.dot(m, w_r[...], precision=_PREC,
                                    preferred_element_type=jnp.float32)
                            + b_r[...])
    ho = _layernorm(h, go_ref[...], beo_ref[...])
    o_ref[0] = (jnp.dot(ho, wo_ref[...], precision=_PREC,
                        preferred_element_type=jnp.float32)
                + bo_ref[...] + xx)


def kernel(x, adj, W1, b1, g1, be1, W2, b2, g2, be2, Wo, bo, go, beo):
    B, T, N, F = x.shape
    BT = B * T
    xr = x.reshape(BT, N, F)

    ar, dv = pl.pallas_call(
        _prep_kernel,
        out_shape=[jax.ShapeDtypeStruct((N, N), jnp.float32),
                   jax.ShapeDtypeStruct((N, 128), jnp.float32)],
    )(adj)

    vec = lambda v: v.reshape(1, -1)
    full = lambda shp: pl.BlockSpec(shp, lambda i: (0,) * len(shp))
    out = pl.pallas_call(
        _gcn_kernel,
        grid=(BT,),
        in_specs=[
            full((N, N)),
            full((N, 128)),
            pl.BlockSpec((1, N, F), lambda i: (i, 0, 0)),
            full((128, 128)), full((1, 128)), full((1, 128)), full((1, 128)),
            full((128, 128)), full((1, 128)), full((1, 128)), full((1, 128)),
            full((128, 128)), full((1, 128)), full((1, 128)), full((1, 128)),
        ],
        out_specs=pl.BlockSpec((1, N, F), lambda i: (i, 0, 0)),
        out_shape=jax.ShapeDtypeStruct((BT, N, F), jnp.float32),
        compiler_params=pltpu.CompilerParams(
            dimension_semantics=("parallel",)),
    )(ar, dv, xr, W1, vec(b1), vec(g1), vec(be1),
      W2, vec(b2), vec(g2), vec(be2),
      Wo, vec(bo), vec(go), vec(beo))
    return out.reshape(B, T, N, F)


# speedup vs baseline: 4.6633x; 4.6633x over previous
import jax
import jax.numpy as jnp
from jax.experimental import pallas as pl
from jax.experimental.pallas import tpu as pltpu

_PREC = jax.lax.Precision.DEFAULT


def _prep_kernel(adj_ref, ar_ref, dv_ref):
    a = adj_ref[...]
    n = a.shape[0]
    row = jax.lax.broadcasted_iota(jnp.int32, (n, n), 0)
    col = jax.lax.broadcasted_iota(jnp.int32, (n, n), 1)
    a = a + jnp.where(row == col, 1.0, 0.0).astype(a.dtype)
    d = jnp.sum(a, axis=1, keepdims=True)
    dinv = jax.lax.rsqrt(d)
    ar_ref[...] = a * dinv
    dv_ref[...] = jnp.broadcast_to(dinv, dv_ref.shape)


def _layernorm(h, g, be, eps=1e-5):
    mu = jnp.mean(h, axis=-1, keepdims=True)
    c = h - mu
    var = jnp.mean(c * c, axis=-1, keepdims=True)
    return c * jax.lax.rsqrt(var + eps) * g + be


def _gcn_kernel(ar_ref, dv_ref, x_ref,
                w1_ref, b1_ref, g1_ref, be1_ref,
                w2_ref, b2_ref, g2_ref, be2_ref,
                wo_ref, bo_ref, go_ref, beo_ref, o_ref):
    ar = ar_ref[...]
    dv = dv_ref[...][:, :x_ref.shape[-1]]
    xx = x_ref[0]
    h = xx
    for (w_r, b_r, g_r, be_r) in ((w1_ref, b1_ref, g1_ref, be1_ref),
                                  (w2_ref, b2_ref, g2_ref, be2_ref)):
        hn = _layernorm(h, g_r[...], be_r[...])
        m = jnp.dot(ar, dv * hn, precision=_PREC,
                    preferred_element_type=jnp.float32)
        h = h + jax.nn.relu(jnp.dot(m, w_r[...], precision=_PREC,
                                    preferred_element_type=jnp.float32)
                            + b_r[...])
    ho = _layernorm(h, go_ref[...], beo_ref[...])
    o_ref[0] = (jnp.dot(ho, wo_ref[...], precision=_PREC,
                        preferred_element_type=jnp.float32)
                + bo_ref[...] + xx)


def kernel(x, adj, W1, b1, g1, be1, W2, b2, g2, be2, Wo, bo, go, beo):
    B, T, N, F = x.shape
    BT = B * T
    xr = x.reshape(BT, N, F)

    ar, dv = pl.pallas_call(
        _prep_kernel,
        out_shape=[jax.ShapeDtypeStruct((N, N), jnp.float32),
                   jax.ShapeDtypeStruct((N, 128), jnp.float32)],
    )(adj)

    vec = lambda v: v.reshape(1, -1)
    full = lambda shp: pl.BlockSpec(shp, lambda i: (0,) * len(shp))
    out = pl.pallas_call(
        _gcn_kernel,
        grid=(BT,),
        in_specs=[
            full((N, N)),
            full((N, 128)),
            pl.BlockSpec((1, N, F), lambda i: (i, 0, 0)),
            full((128, 128)), full((1, 128)), full((1, 128)), full((1, 128)),
            full((128, 128)), full((1, 128)), full((1, 128)), full((1, 128)),
            full((128, 128)), full((1, 128)), full((1, 128)), full((1, 128)),
        ],
        out_specs=pl.BlockSpec((1, N, F), lambda i: (i, 0, 0)),
        out_shape=jax.ShapeDtypeStruct((BT, N, F), jnp.float32),
        compiler_params=pltpu.CompilerParams(
            dimension_semantics=("parallel",)),
    )(ar, dv, xr, W1, vec(b1), vec(g1), vec(be1),
      W2, vec(b2), vec(g2), vec(be2),
      Wo, vec(bo), vec(go), vec(beo))
    return out.reshape(B, T, N, F)
